# baseline (speedup 1.0000x reference)
import jax
import jax.numpy as jnp
from jax.experimental import pallas as pl
from jax.experimental.pallas import tpu as pltpu

_B, _D, _N = 16, 256, 1024
_NE = 1024


def _vq_body(x_ref, e_ref, et_ref, aux_ref, q_ref, idx_ref, loss_ref, h_ref):
    b = pl.program_id(0)
    x = x_ref[0]
    e = e_ref[...]
    et = et_ref[...]

    scores = jax.lax.dot_general(
        et, x, (((1,), (0,)), ((), ())),
        preferred_element_type=jnp.float32)
    e_sq = jnp.sum(e * e, axis=0, keepdims=True).reshape(_NE, 1)
    x_sq = jnp.sum(x * x, axis=0, keepdims=True)
    dist = (x_sq + e_sq) - 2.0 * scores

    minval = jnp.min(dist, axis=0, keepdims=True)
    h_ref[...] = jnp.where(dist == minval, 1.0, 0.0)

    aux = jax.lax.dot_general(
        aux_ref[...], h_ref[...], (((1,), (0,)), ((), ())),
        preferred_element_type=jnp.float32)
    idx_ref[0] = (aux[1:2] * 32.0 + aux[2:3]).astype(jnp.int32)

    tie = jnp.max(aux[0:1]) > 1.5

    @pl.when(tie)
    def _():
        iota = jax.lax.broadcasted_iota(jnp.int32, (_NE, _N), 0)
        idxe = jnp.min(jnp.where(dist == minval, iota, _NE), axis=0,
                       keepdims=True)
        idx_ref[0] = idxe
        h_ref[...] = jnp.where(iota == idxe, 1.0, 0.0)

    q = jax.lax.dot_general(
        e, h_ref[...], (((1,), (0,)), ((), ())),
        preferred_element_type=jnp.float32)
    q_ref[0] = q

    part = jnp.sum(minval)

    @pl.when(b == 0)
    def _():
        loss_ref[0, 0] = part

    @pl.when(b > 0)
    def _():
        loss_ref[0, 0] = loss_ref[0, 0] + part


def kernel(inputs, embedding):
    emb_t = embedding.T
    j = jnp.arange(_NE, dtype=jnp.float32)
    aux_rows = jnp.stack([
        jnp.ones((_NE,), jnp.float32),
        jnp.floor(j / 32.0),
        j % 32.0,
    ] + [jnp.zeros((_NE,), jnp.float32)] * 5, axis=0)

    q, idx, loss_sum = pl.pallas_call(
        _vq_body,
        grid=(_B,),
        in_specs=[
            pl.BlockSpec((1, _D, _N), lambda b: (b, 0, 0)),
            pl.BlockSpec((_D, _NE), lambda b: (0, 0)),
            pl.BlockSpec((_NE, _D), lambda b: (0, 0)),
            pl.BlockSpec((8, _NE), lambda b: (0, 0)),
        ],
        out_specs=[
            pl.BlockSpec((1, _D, _N), lambda b: (b, 0, 0)),
            pl.BlockSpec((1, 1, _N), lambda b: (b, 0, 0)),
            pl.BlockSpec((1, 1), lambda b: (0, 0), memory_space=pltpu.SMEM),
        ],
        out_shape=[
            jax.ShapeDtypeStruct((_B, _D, _N), jnp.float32),
            jax.ShapeDtypeStruct((_B, 1, _N), jnp.int32),
            jax.ShapeDtypeStruct((1, 1), jnp.float32),
        ],
        scratch_shapes=[pltpu.VMEM((_NE, _N), jnp.float32)],
    )(inputs, embedding, emb_t, aux_rows)

    loss = loss_sum[0, 0] / jnp.float32(_B * _D * _N)
    return (q, idx.reshape(_B, _N), loss)


# speedup vs baseline: 1.1932x; 1.1932x over previous
import jax
import jax.numpy as jnp
from jax.experimental import pallas as pl
from jax.experimental.pallas import tpu as pltpu

_B, _D, _N = 16, 256, 1024
_NE = 1024


def _vq_body(x_ref, e_ref, et_ref, aux_ref, q_ref, idx_ref, loss_ref):
    b = pl.program_id(0)
    x = x_ref[0]
    e = e_ref[...]
    et = et_ref[...]

    scores = jax.lax.dot_general(
        et, x, (((1,), (0,)), ((), ())),
        preferred_element_type=jnp.float32)
    e_sq = jnp.sum(e * e, axis=0, keepdims=True).reshape(_NE, 1)
    x_sq = jnp.sum(x * x, axis=0, keepdims=True)
    dist = (x_sq + e_sq) - 2.0 * scores

    minval = jnp.min(dist, axis=0, keepdims=True)
    h = jnp.where(dist == minval, 1.0, 0.0)

    aux = jax.lax.dot_general(
        aux_ref[...], h, (((1,), (0,)), ((), ())),
        preferred_element_type=jnp.float32)
    idx_ref[0] = (aux[1:2] * 32.0 + aux[2:3]).astype(jnp.int32)
    q_ref[0] = jax.lax.dot_general(
        e, h, (((1,), (0,)), ((), ())),
        preferred_element_type=jnp.float32)

    tie = jnp.max(aux[0:1]) > 1.5

    @pl.when(tie)
    def _():
        iota = jax.lax.broadcasted_iota(jnp.int32, (_NE, _N), 0)
        idxe = jnp.min(jnp.where(dist == minval, iota, _NE), axis=0,
                       keepdims=True)
        idx_ref[0] = idxe
        he = jnp.where(iota == idxe, 1.0, 0.0)
        q_ref[0] = jax.lax.dot_general(
            e, he, (((1,), (0,)), ((), ())),
            preferred_element_type=jnp.float32)

    part = jnp.sum(minval)

    @pl.when(b == 0)
    def _():
        loss_ref[0, 0] = part

    @pl.when(b > 0)
    def _():
        loss_ref[0, 0] = loss_ref[0, 0] + part


def kernel(inputs, embedding):
    emb_t = embedding.T
    j = jnp.arange(_NE, dtype=jnp.float32)
    aux_rows = jnp.stack([
        jnp.ones((_NE,), jnp.float32),
        jnp.floor(j / 32.0),
        j % 32.0,
    ] + [jnp.zeros((_NE,), jnp.float32)] * 5, axis=0)

    q, idx, loss_sum = pl.pallas_call(
        _vq_body,
        grid=(_B,),
        in_specs=[
            pl.BlockSpec((1, _D, _N), lambda b: (b, 0, 0)),
            pl.BlockSpec((_D, _NE), lambda b: (0, 0)),
            pl.BlockSpec((_NE, _D), lambda b: (0, 0)),
            pl.BlockSpec((8, _NE), lambda b: (0, 0)),
        ],
        out_specs=[
            pl.BlockSpec((1, _D, _N), lambda b: (b, 0, 0)),
            pl.BlockSpec((1, 1, _N), lambda b: (b, 0, 0)),
            pl.BlockSpec((1, 1), lambda b: (0, 0), memory_space=pltpu.SMEM),
        ],
        out_shape=[
            jax.ShapeDtypeStruct((_B, _D, _N), jnp.float32),
            jax.ShapeDtypeStruct((_B, 1, _N), jnp.int32),
            jax.ShapeDtypeStruct((1, 1), jnp.float32),
        ],
    )(inputs, embedding, emb_t, aux_rows)

    loss = loss_sum[0, 0] / jnp.float32(_B * _D * _N)
    return (q, idx.reshape(_B, _N), loss)
